# Initial kernel scaffold; baseline (speedup 1.0000x reference)
import jax
import jax.numpy as jnp
from jax.experimental import pallas as pl


def kernel(raw_cls_bbox, raw_prob):
    raise NotImplementedError("write your pallas kernel here")


# speedup vs baseline: 1.2266x; 1.2266x over previous
import functools

import jax
import jax.numpy as jnp
from jax import lax
from jax.experimental import pallas as pl

_C = 20
_N = 20000
_NPAD = 20480
_ROWS = 160
_LANES = 128
_K = 1024
_TOPK = 1000
_NMS_T = 0.3
_SCORE_T = 0.05
_NCHUNK = 10
_CHUNK = 2048

_HI = jax.lax.Precision.HIGHEST


def _select_kernel(s_ref, pos_ref):
    s = s_ref[...]
    m = jnp.where(s > _SCORE_T, s, 0.0)
    key = lax.bitcast_convert_type(m, jnp.int32)
    ii = (lax.broadcasted_iota(jnp.int32, (_C, _ROWS, _LANES), 1) * _LANES
          + lax.broadcasted_iota(jnp.int32, (_C, _ROWS, _LANES), 2))

    def bs_body(_, c):
        lo, hi = c
        mid = lo + (hi - lo) // 2
        cnt = jnp.sum((key >= mid).astype(jnp.int32), axis=(1, 2),
                      keepdims=True)
        pred = cnt >= _K
        return jnp.where(pred, mid, lo), jnp.where(pred, hi, mid)

    lo0 = jnp.zeros((_C, 1, 1), jnp.int32)
    hi0 = jnp.full((_C, 1, 1), 0x7FFFFFFF, jnp.int32)
    vstar, _ = lax.fori_loop(0, 31, bs_body, (lo0, hi0))

    c_gt = jnp.sum((key > vstar).astype(jnp.int32), axis=(1, 2),
                   keepdims=True)
    need_eq = _K - c_gt
    eq = key == vstar

    def ts_body(_, c):
        lo, hi = c
        mid = lo + (hi - lo) // 2
        cnt = jnp.sum((eq & (ii < mid)).astype(jnp.int32), axis=(1, 2),
                      keepdims=True)
        pred = cnt >= need_eq
        return jnp.where(pred, lo, mid), jnp.where(pred, mid, hi)

    lo0 = jnp.zeros((_C, 1, 1), jnp.int32)
    hi0 = jnp.full((_C, 1, 1), _NPAD, jnp.int32)
    _, tstar = lax.fori_loop(0, 15, ts_body, (lo0, hi0))

    mask = (key > vstar) | (eq & (ii < tstar))
    mf = mask.astype(jnp.float32)

    u128 = (lax.broadcasted_iota(jnp.int32, (_LANES, _LANES), 0)
            < lax.broadcasted_iota(jnp.int32, (_LANES, _LANES), 1))
    lane_ex = lax.dot_general(mf.reshape(_C * _ROWS, _LANES),
                              u128.astype(jnp.float32),
                              (((1,), (0,)), ((), ())), precision=_HI,
                              preferred_element_type=jnp.float32)
    lane_ex = lane_ex.reshape(_C, _ROWS, _LANES)
    rowsum = jnp.sum(mf, axis=2, keepdims=True)
    rs = jnp.concatenate(
        [jnp.zeros((_C, 1, 1), jnp.float32), rowsum[:, :-1, :]], axis=1)
    d = 1
    while d < _ROWS:
        shifted = jnp.concatenate(
            [jnp.zeros((_C, d, 1), jnp.float32), rs[:, :-d, :]], axis=1)
        rs = rs + shifted
        d *= 2
    pos = lane_ex + rs
    pos_ref[...] = jnp.where(mask, pos, -1.0)


def _compact_kernel(pos_ref, s_ref, b_ref, out_ref):
    j = pl.program_id(1)
    pos = pos_ref[...].reshape(1, _CHUNK)
    s = s_ref[...].reshape(1, _CHUNK)
    m = jnp.where(s > _SCORE_T, s, 0.0)
    b4 = b_ref[...].reshape(4, _CHUNK)
    payload = jnp.concatenate([m, b4], axis=0)
    iota_p = lax.broadcasted_iota(jnp.int32, (_K, _CHUNK), 0)
    onehot = (iota_p == pos.astype(jnp.int32)).astype(jnp.float32)
    contrib = lax.dot_general(payload, onehot, (((1,), (1,)), ((), ())),
                              precision=_HI,
                              preferred_element_type=jnp.float32)
    contrib = jnp.concatenate(
        [contrib, jnp.zeros((3, _K), jnp.float32)], axis=0)

    @pl.when(j == 0)
    def _():
        out_ref[...] = jnp.zeros_like(out_ref)

    out_ref[...] += contrib.reshape(1, 8, _K)


def _nms_kernel(a_ref, at_ref, boxes_ref, scores_ref, labels_ref, keep_ref):
    a = a_ref[...].reshape(8, _K)
    at = at_ref[...].reshape(_K, 8)
    m_row = a[0:1, :]
    m_col = at[:, 0:1]
    q_row = lax.broadcasted_iota(jnp.int32, (1, _K), 1)
    p_col = lax.broadcasted_iota(jnp.int32, (_K, 1), 0)

    cmp = (m_row > m_col) | ((m_row == m_col) & (q_row < p_col))
    rank = jnp.sum(cmp.astype(jnp.int32), axis=1, keepdims=True)
    r_row = lax.broadcasted_iota(jnp.int32, (1, _K), 1)
    onehot2 = (rank == r_row).astype(jnp.float32)

    sorted_row = lax.dot_general(a[0:5, :], onehot2, (((1,), (0,)), ((), ())),
                                 precision=_HI,
                                 preferred_element_type=jnp.float32)
    sorted_col = lax.dot_general(onehot2, at[:, 0:5],
                                 (((0,), (0,)), ((), ())), precision=_HI,
                                 preferred_element_type=jnp.float32)

    ms_row = sorted_row[0:1, :]
    y1r, x1r = sorted_row[1:2, :], sorted_row[2:3, :]
    y2r, x2r = sorted_row[3:4, :], sorted_row[4:5, :]
    y1c, x1c = sorted_col[:, 1:2], sorted_col[:, 2:3]
    y2c, x2c = sorted_col[:, 3:4], sorted_col[:, 4:5]

    r_i32 = lax.broadcasted_iota(jnp.int32, (1, _K), 1)
    valid = (ms_row > _SCORE_T) & (r_i32 < _TOPK)

    yy1 = jnp.maximum(y1c, y1r)
    xx1 = jnp.maximum(x1c, x1r)
    yy2 = jnp.minimum(y2c, y2r)
    xx2 = jnp.minimum(x2c, x2r)
    inter = (jnp.clip(yy2 - yy1, 0.0) * jnp.clip(xx2 - xx1, 0.0))
    area_c = (y2c - y1c) * (x2c - x1c)
    area_r = (y2r - y1r) * (x2r - x1r)
    union = area_c + area_r - inter
    iou = inter / jnp.maximum(union, 1e-9)

    i_col = lax.broadcasted_iota(jnp.int32, (_K, _K), 0)
    j_row = lax.broadcasted_iota(jnp.int32, (_K, _K), 1)
    sup = ((iou > _NMS_T) & (j_row > i_col)).astype(jnp.float32)

    valid_f = valid.astype(jnp.float32)

    def cond(c):
        return jnp.logical_not(c[1])

    def body(c):
        k, _ = c
        s = lax.dot_general(k, sup, (((1,), (0,)), ((), ())), precision=_HI,
                            preferred_element_type=jnp.float32)
        k_new = jnp.where((s == 0.0), valid_f, 0.0)
        done = jnp.sum(jnp.abs(k_new - k)) == 0.0
        return k_new, done

    k_fix, _ = lax.while_loop(cond, body, (valid_f, jnp.bool_(False)))
    keep = k_fix > 0.0

    cls = pl.program_id(0)
    boxes_ref[...] = sorted_col[:_TOPK, 1:5].reshape(1, _TOPK, 4)
    scores_ref[...] = jnp.where(keep, ms_row, 0.0)[:, :_TOPK].reshape(
        1, 1, _TOPK)
    labels_ref[...] = jnp.full((1, 1, _TOPK), cls, jnp.int32)
    keep_ref[...] = keep[:, :_TOPK].astype(jnp.int32).reshape(1, 1, _TOPK)


@jax.jit
def kernel(raw_cls_bbox, raw_prob):
    scores_t = jnp.pad(raw_prob[:, 1:].T, ((0, 0), (0, _NPAD - _N)))
    scores3 = scores_t.reshape(_C, _ROWS, _LANES)
    boxes = raw_cls_bbox.reshape(_N, _C + 1, 4)[:, 1:, :]
    boxes_t = jnp.pad(jnp.transpose(boxes, (1, 2, 0)),
                      ((0, 0), (0, 0), (0, _NPAD - _N)))
    boxes_c = jnp.transpose(boxes_t.reshape(_C, 4, _NCHUNK, _CHUNK),
                            (0, 2, 1, 3))

    pos = pl.pallas_call(
        _select_kernel,
        out_shape=jax.ShapeDtypeStruct((_C, _ROWS, _LANES), jnp.float32),
    )(scores3)

    pos_c = pos.reshape(_C, _NCHUNK, 1, _CHUNK)
    scores_c = scores_t.reshape(_C, _NCHUNK, 1, _CHUNK)

    a8 = pl.pallas_call(
        _compact_kernel,
        grid=(_C, _NCHUNK),
        in_specs=[
            pl.BlockSpec((1, 1, 1, _CHUNK), lambda c, j: (c, j, 0, 0)),
            pl.BlockSpec((1, 1, 1, _CHUNK), lambda c, j: (c, j, 0, 0)),
            pl.BlockSpec((1, 1, 4, _CHUNK), lambda c, j: (c, j, 0, 0)),
        ],
        out_specs=pl.BlockSpec((1, 8, _K), lambda c, j: (c, 0, 0)),
        out_shape=jax.ShapeDtypeStruct((_C, 8, _K), jnp.float32),
    )(pos_c, scores_c, boxes_c)

    a8_t = jnp.transpose(a8, (0, 2, 1))

    top_boxes, out_scores, labels, keep_i32 = pl.pallas_call(
        _nms_kernel,
        grid=(_C,),
        in_specs=[
            pl.BlockSpec((1, 8, _K), lambda c: (c, 0, 0)),
            pl.BlockSpec((1, _K, 8), lambda c: (c, 0, 0)),
        ],
        out_specs=[
            pl.BlockSpec((1, _TOPK, 4), lambda c: (c, 0, 0)),
            pl.BlockSpec((1, 1, _TOPK), lambda c: (c, 0, 0)),
            pl.BlockSpec((1, 1, _TOPK), lambda c: (c, 0, 0)),
            pl.BlockSpec((1, 1, _TOPK), lambda c: (c, 0, 0)),
        ],
        out_shape=[
            jax.ShapeDtypeStruct((_C, _TOPK, 4), jnp.float32),
            jax.ShapeDtypeStruct((_C, 1, _TOPK), jnp.float32),
            jax.ShapeDtypeStruct((_C, 1, _TOPK), jnp.int32),
            jax.ShapeDtypeStruct((_C, 1, _TOPK), jnp.int32),
        ],
    )(a8, a8_t)

    return (top_boxes, out_scores.reshape(_C, _TOPK),
            labels.reshape(_C, _TOPK),
            keep_i32.reshape(_C, _TOPK).astype(bool))


# speedup vs baseline: 4.0466x; 3.2992x over previous
import functools

import jax
import jax.numpy as jnp
from jax import lax
from jax.experimental import pallas as pl
from jax.experimental.pallas import tpu as pltpu
from jax.experimental.pallas import tpu_sc as plsc

_C = 20
_N = 20000
_NPAD = 20480
_ROWS = 160
_LANES = 128
_K = 1024
_TOPK = 1000
_NMS_T = 0.3
_SCORE_T = 0.05
_NCHUNK = 10
_CHUNK = 2048

_HI = jax.lax.Precision.HIGHEST


def _select_kernel(s_ref, pos_ref):
    s = s_ref[...]
    m = jnp.where(s > _SCORE_T, s, 0.0)
    key = lax.bitcast_convert_type(m, jnp.int32)
    ii = (lax.broadcasted_iota(jnp.int32, (_C, _ROWS, _LANES), 1) * _LANES
          + lax.broadcasted_iota(jnp.int32, (_C, _ROWS, _LANES), 2))

    def bs_body(_, c):
        lo, hi = c
        mid = lo + (hi - lo) // 2
        cnt = jnp.sum((key >= mid).astype(jnp.int32), axis=(1, 2),
                      keepdims=True)
        pred = cnt >= _K
        return jnp.where(pred, mid, lo), jnp.where(pred, hi, mid)

    lo0 = jnp.zeros((_C, 1, 1), jnp.int32)
    hi0 = jnp.full((_C, 1, 1), 0x7FFFFFFF, jnp.int32)
    vstar, _ = lax.fori_loop(0, 31, bs_body, (lo0, hi0))

    c_gt = jnp.sum((key > vstar).astype(jnp.int32), axis=(1, 2),
                   keepdims=True)
    need_eq = _K - c_gt
    eq = key == vstar

    def ts_body(_, c):
        lo, hi = c
        mid = lo + (hi - lo) // 2
        cnt = jnp.sum((eq & (ii < mid)).astype(jnp.int32), axis=(1, 2),
                      keepdims=True)
        pred = cnt >= need_eq
        return jnp.where(pred, lo, mid), jnp.where(pred, mid, hi)

    lo0 = jnp.zeros((_C, 1, 1), jnp.int32)
    hi0 = jnp.full((_C, 1, 1), _NPAD, jnp.int32)
    _, tstar = lax.fori_loop(0, 15, ts_body, (lo0, hi0))

    mask = (key > vstar) | (eq & (ii < tstar))
    mf = mask.astype(jnp.float32)

    u128 = (lax.broadcasted_iota(jnp.int32, (_LANES, _LANES), 0)
            < lax.broadcasted_iota(jnp.int32, (_LANES, _LANES), 1))
    lane_ex = lax.dot_general(mf.reshape(_C * _ROWS, _LANES),
                              u128.astype(jnp.float32),
                              (((1,), (0,)), ((), ())), precision=_HI,
                              preferred_element_type=jnp.float32)
    lane_ex = lane_ex.reshape(_C, _ROWS, _LANES)
    rowsum = jnp.sum(mf, axis=2, keepdims=True)
    rs = jnp.concatenate(
        [jnp.zeros((_C, 1, 1), jnp.float32), rowsum[:, :-1, :]], axis=1)
    d = 1
    while d < _ROWS:
        shifted = jnp.concatenate(
            [jnp.zeros((_C, d, 1), jnp.float32), rs[:, :-d, :]], axis=1)
        rs = rs + shifted
        d *= 2
    pos = lane_ex + rs
    pos_ref[...] = jnp.where(mask, pos.astype(jnp.int32), -1)


_NSLICE = _NPAD // 16
_SC_MESH = plsc.VectorSubcoreMesh(core_axis_name="c", subcore_axis_name="s")


@functools.partial(
    pl.kernel,
    mesh=_SC_MESH,
    out_type=jax.ShapeDtypeStruct((_C, 6, _K), jnp.float32),
    compiler_params=pltpu.CompilerParams(needs_layout_passes=False),
    scratch_types=[
        pltpu.VMEM((_NPAD,), jnp.int32),
        pltpu.VMEM((_NPAD,), jnp.float32),
        pltpu.VMEM((_K,), jnp.float32),
        pltpu.VMEM((_K,), jnp.float32),
        pltpu.VMEM((_K,), jnp.float32),
        pltpu.VMEM((_K,), jnp.float32),
        pltpu.VMEM((_K,), jnp.float32),
    ],
)
def _sc_compact(pos_hbm, s_hbm, b_hbm, out_hbm, pos_v, val_v,
                a0, a1, a2, a3, a4):
    wid = lax.axis_index("s") * 2 + lax.axis_index("c")
    outs = (a0, a1, a2, a3, a4)

    @pl.when(wid < _C)
    def _():
        pltpu.sync_copy(pos_hbm.at[wid], pos_v)
        for k in range(5):
            if k == 0:
                pltpu.sync_copy(s_hbm.at[wid], val_v)
            else:
                pltpu.sync_copy(b_hbm.at[wid, k - 1], val_v)

            def body(i, carry, k=k):
                idx = pos_v[pl.ds(i * 16, 16)]
                v16 = val_v[pl.ds(i * 16, 16)]
                if k == 0:
                    v16 = jnp.where(v16 > _SCORE_T, v16, 0.0)
                plsc.store_scatter(outs[k], [idx], v16, mask=idx >= 0)
                return carry

            lax.fori_loop(0, _NSLICE, body, 0)
        for k in range(5):
            pltpu.sync_copy(outs[k], out_hbm.at[wid, k])


def _nms_kernel(a_ref, at_ref, boxes_ref, scores_ref, labels_ref, keep_ref):
    a = a_ref[...].reshape(6, _K)
    at = at_ref[...].reshape(_K, 6)
    m_row = a[0:1, :]
    m_col = at[:, 0:1]
    q_row = lax.broadcasted_iota(jnp.int32, (1, _K), 1)
    p_col = lax.broadcasted_iota(jnp.int32, (_K, 1), 0)

    cmp = (m_row > m_col) | ((m_row == m_col) & (q_row < p_col))
    rank = jnp.sum(cmp.astype(jnp.int32), axis=1, keepdims=True)
    r_row = lax.broadcasted_iota(jnp.int32, (1, _K), 1)
    onehot2 = (rank == r_row).astype(jnp.float32)

    sorted_row = lax.dot_general(a[0:5, :], onehot2, (((1,), (0,)), ((), ())),
                                 precision=_HI,
                                 preferred_element_type=jnp.float32)
    sorted_col = lax.dot_general(onehot2, at[:, 0:5],
                                 (((0,), (0,)), ((), ())), precision=_HI,
                                 preferred_element_type=jnp.float32)

    ms_row = sorted_row[0:1, :]
    y1r, x1r = sorted_row[1:2, :], sorted_row[2:3, :]
    y2r, x2r = sorted_row[3:4, :], sorted_row[4:5, :]
    y1c, x1c = sorted_col[:, 1:2], sorted_col[:, 2:3]
    y2c, x2c = sorted_col[:, 3:4], sorted_col[:, 4:5]

    r_i32 = lax.broadcasted_iota(jnp.int32, (1, _K), 1)
    valid = (ms_row > _SCORE_T) & (r_i32 < _TOPK)

    yy1 = jnp.maximum(y1c, y1r)
    xx1 = jnp.maximum(x1c, x1r)
    yy2 = jnp.minimum(y2c, y2r)
    xx2 = jnp.minimum(x2c, x2r)
    inter = (jnp.clip(yy2 - yy1, 0.0) * jnp.clip(xx2 - xx1, 0.0))
    area_c = (y2c - y1c) * (x2c - x1c)
    area_r = (y2r - y1r) * (x2r - x1r)
    union = area_c + area_r - inter
    iou = inter / jnp.maximum(union, 1e-9)

    i_col = lax.broadcasted_iota(jnp.int32, (_K, _K), 0)
    j_row = lax.broadcasted_iota(jnp.int32, (_K, _K), 1)
    sup = ((iou > _NMS_T) & (j_row > i_col)).astype(jnp.float32)

    valid_f = valid.astype(jnp.float32)

    def cond(c):
        return jnp.logical_not(c[1])

    def body(c):
        k, _ = c
        s = lax.dot_general(k, sup, (((1,), (0,)), ((), ())), precision=_HI,
                            preferred_element_type=jnp.float32)
        k_new = jnp.where((s == 0.0), valid_f, 0.0)
        done = jnp.sum(jnp.abs(k_new - k)) == 0.0
        return k_new, done

    k_fix, _ = lax.while_loop(cond, body, (valid_f, jnp.bool_(False)))
    keep = k_fix > 0.0

    cls = pl.program_id(0)
    boxes_ref[...] = sorted_col[:_TOPK, 1:5].reshape(1, _TOPK, 4)
    scores_ref[...] = jnp.where(keep, ms_row, 0.0)[:, :_TOPK].reshape(
        1, 1, _TOPK)
    labels_ref[...] = jnp.full((1, 1, _TOPK), cls, jnp.int32)
    keep_ref[...] = keep[:, :_TOPK].astype(jnp.int32).reshape(1, 1, _TOPK)


@jax.jit
def kernel(raw_cls_bbox, raw_prob):
    scores_t = jnp.pad(raw_prob[:, 1:].T, ((0, 0), (0, _NPAD - _N)))
    scores3 = scores_t.reshape(_C, _ROWS, _LANES)
    boxes = raw_cls_bbox.reshape(_N, _C + 1, 4)[:, 1:, :]
    boxes_t = jnp.pad(jnp.transpose(boxes, (1, 2, 0)),
                      ((0, 0), (0, 0), (0, _NPAD - _N)))

    pos = pl.pallas_call(
        _select_kernel,
        out_shape=jax.ShapeDtypeStruct((_C, _ROWS, _LANES), jnp.int32),
    )(scores3)

    a6 = _sc_compact(pos.reshape(_C, _NPAD), scores_t, boxes_t)
    a6_t = jnp.transpose(a6, (0, 2, 1))

    top_boxes, out_scores, labels, keep_i32 = pl.pallas_call(
        _nms_kernel,
        grid=(_C,),
        in_specs=[
            pl.BlockSpec((1, 6, _K), lambda c: (c, 0, 0)),
            pl.BlockSpec((1, _K, 6), lambda c: (c, 0, 0)),
        ],
        out_specs=[
            pl.BlockSpec((1, _TOPK, 4), lambda c: (c, 0, 0)),
            pl.BlockSpec((1, 1, _TOPK), lambda c: (c, 0, 0)),
            pl.BlockSpec((1, 1, _TOPK), lambda c: (c, 0, 0)),
            pl.BlockSpec((1, 1, _TOPK), lambda c: (c, 0, 0)),
        ],
        out_shape=[
            jax.ShapeDtypeStruct((_C, _TOPK, 4), jnp.float32),
            jax.ShapeDtypeStruct((_C, 1, _TOPK), jnp.float32),
            jax.ShapeDtypeStruct((_C, 1, _TOPK), jnp.int32),
            jax.ShapeDtypeStruct((_C, 1, _TOPK), jnp.int32),
        ],
    )(a6, a6_t)

    return (top_boxes, out_scores.reshape(_C, _TOPK),
            labels.reshape(_C, _TOPK),
            keep_i32.reshape(_C, _TOPK).astype(bool))


# speedup vs baseline: 5.9544x; 1.4714x over previous
import functools

import jax
import jax.numpy as jnp
from jax import lax
from jax.experimental import pallas as pl
from jax.experimental.pallas import tpu as pltpu
from jax.experimental.pallas import tpu_sc as plsc

_C = 20
_N = 20000
_NPAD = 20480
_ROWS = 160
_LANES = 128
_K = 1024
_TOPK = 1000
_NMS_T = 0.3
_SCORE_T = 0.05
_NCHUNK = 10
_CHUNK = 2048

_HI = jax.lax.Precision.HIGHEST


def _select_kernel(s_ref, pos_ref):
    s = s_ref[...]
    m = jnp.where(s > _SCORE_T, s, 0.0)
    key = lax.bitcast_convert_type(m, jnp.int32)
    ii = (lax.broadcasted_iota(jnp.int32, (_C, _ROWS, _LANES), 1) * _LANES
          + lax.broadcasted_iota(jnp.int32, (_C, _ROWS, _LANES), 2))

    def bs_body(_, c):
        lo, hi = c
        mid = lo + (hi - lo) // 2
        cnt = jnp.sum((key >= mid).astype(jnp.int32), axis=(1, 2),
                      keepdims=True)
        pred = cnt >= _K
        return jnp.where(pred, mid, lo), jnp.where(pred, hi, mid)

    lo0 = jnp.zeros((_C, 1, 1), jnp.int32)
    hi0 = jnp.full((_C, 1, 1), 0x7FFFFFFF, jnp.int32)
    vstar, _ = lax.fori_loop(0, 31, bs_body, (lo0, hi0))

    c_gt = jnp.sum((key > vstar).astype(jnp.int32), axis=(1, 2),
                   keepdims=True)
    need_eq = _K - c_gt
    eq = key == vstar

    def ts_body(_, c):
        lo, hi = c
        mid = lo + (hi - lo) // 2
        cnt = jnp.sum((eq & (ii < mid)).astype(jnp.int32), axis=(1, 2),
                      keepdims=True)
        pred = cnt >= need_eq
        return jnp.where(pred, lo, mid), jnp.where(pred, mid, hi)

    lo0 = jnp.zeros((_C, 1, 1), jnp.int32)
    hi0 = jnp.full((_C, 1, 1), _NPAD, jnp.int32)
    _, tstar = lax.fori_loop(0, 15, ts_body, (lo0, hi0))

    mask = (key > vstar) | (eq & (ii < tstar))
    mf = mask.astype(jnp.float32)

    u128 = (lax.broadcasted_iota(jnp.int32, (_LANES, _LANES), 0)
            < lax.broadcasted_iota(jnp.int32, (_LANES, _LANES), 1))
    lane_ex = lax.dot_general(mf.reshape(_C * _ROWS, _LANES),
                              u128.astype(jnp.float32),
                              (((1,), (0,)), ((), ())), precision=_HI,
                              preferred_element_type=jnp.float32)
    lane_ex = lane_ex.reshape(_C, _ROWS, _LANES)
    rowsum = jnp.sum(mf, axis=2, keepdims=True)
    rs = jnp.concatenate(
        [jnp.zeros((_C, 1, 1), jnp.float32), rowsum[:, :-1, :]], axis=1)
    d = 1
    while d < _ROWS:
        shifted = jnp.concatenate(
            [jnp.zeros((_C, d, 1), jnp.float32), rs[:, :-d, :]], axis=1)
        rs = rs + shifted
        d *= 2
    pos = lane_ex + rs
    pos_ref[...] = jnp.where(mask, pos.astype(jnp.int32), -1)


_NSLICE = _NPAD // 16
_SC_MESH = plsc.VectorSubcoreMesh(core_axis_name="c", subcore_axis_name="s")


@functools.partial(
    pl.kernel,
    mesh=_SC_MESH,
    out_type=jax.ShapeDtypeStruct((_C, 6, _K), jnp.float32),
    compiler_params=pltpu.CompilerParams(needs_layout_passes=False),
    scratch_types=[
        pltpu.VMEM((_NPAD,), jnp.int32),
        pltpu.VMEM((_NPAD,), jnp.float32),
        pltpu.VMEM((_K,), jnp.float32),
        pltpu.VMEM((_K,), jnp.float32),
        pltpu.VMEM((_K,), jnp.float32),
        pltpu.VMEM((_K,), jnp.float32),
        pltpu.VMEM((_K,), jnp.float32),
    ],
)
def _sc_compact(pos_hbm, s_hbm, b_hbm, out_hbm, pos_v, val_v,
                a0, a1, a2, a3, a4):
    wid = lax.axis_index("s") * 2 + lax.axis_index("c")
    outs = (a0, a1, a2, a3, a4)

    @pl.when(wid < _C)
    def _():
        pltpu.sync_copy(pos_hbm.at[wid], pos_v)
        for k in range(5):
            if k == 0:
                pltpu.sync_copy(s_hbm.at[wid], val_v)
            else:
                pltpu.sync_copy(b_hbm.at[wid, k - 1], val_v)

            def body(i, carry, k=k):
                idx = pos_v[pl.ds(i * 16, 16)]
                v16 = val_v[pl.ds(i * 16, 16)]
                if k == 0:
                    v16 = jnp.where(v16 > _SCORE_T, v16, 0.0)
                plsc.store_scatter(outs[k], [idx], v16, mask=idx >= 0)
                return carry

            lax.fori_loop(0, _NSLICE, body, 0)
        for k in range(5):
            pltpu.sync_copy(outs[k], out_hbm.at[wid, k])


def _nms_kernel(a_ref, at_ref, boxes_ref, labels_ref, msort_ref, valid_ref,
                s8_ref):
    a = a_ref[...].reshape(6, _K)
    at = at_ref[...].reshape(_K, 6)
    m_row = a[0:1, :]
    m_col = at[:, 0:1]
    q_row = lax.broadcasted_iota(jnp.int32, (1, _K), 1)
    p_col = lax.broadcasted_iota(jnp.int32, (_K, 1), 0)

    cmp = (m_row > m_col) | ((m_row == m_col) & (q_row < p_col))
    rank = jnp.sum(cmp.astype(jnp.int32), axis=1, keepdims=True)
    r_row = lax.broadcasted_iota(jnp.int32, (1, _K), 1)
    onehot2 = (rank == r_row).astype(jnp.float32)

    sorted_row = lax.dot_general(a[0:5, :], onehot2, (((1,), (0,)), ((), ())),
                                 precision=_HI,
                                 preferred_element_type=jnp.float32)
    sorted_col = lax.dot_general(onehot2, at[:, 0:5],
                                 (((0,), (0,)), ((), ())), precision=_HI,
                                 preferred_element_type=jnp.float32)

    ms_row = sorted_row[0:1, :]
    y1r, x1r = sorted_row[1:2, :], sorted_row[2:3, :]
    y2r, x2r = sorted_row[3:4, :], sorted_row[4:5, :]
    y1c, x1c = sorted_col[:, 1:2], sorted_col[:, 2:3]
    y2c, x2c = sorted_col[:, 3:4], sorted_col[:, 4:5]

    r_i32 = lax.broadcasted_iota(jnp.int32, (1, _K), 1)
    valid = (ms_row > _SCORE_T) & (r_i32 < _TOPK)

    yy1 = jnp.maximum(y1c, y1r)
    xx1 = jnp.maximum(x1c, x1r)
    yy2 = jnp.minimum(y2c, y2r)
    xx2 = jnp.minimum(x2c, x2r)
    inter = (jnp.clip(yy2 - yy1, 0.0) * jnp.clip(xx2 - xx1, 0.0))
    area_c = (y2c - y1c) * (x2c - x1c)
    area_r = (y2r - y1r) * (x2r - x1r)
    union = area_c + area_r - inter
    iou = inter / jnp.maximum(union, 1e-9)

    i_col = lax.broadcasted_iota(jnp.int32, (_K, _K), 0)
    j_row = lax.broadcasted_iota(jnp.int32, (_K, _K), 1)
    sup = ((iou > _NMS_T) & (j_row > i_col)).astype(jnp.int8)

    cls = pl.program_id(0)
    boxes_ref[...] = sorted_col[:_TOPK, 1:5].reshape(1, _TOPK, 4)
    labels_ref[...] = jnp.full((1, 1, _TOPK), cls, jnp.int32)
    msort_ref[...] = ms_row.reshape(1, 1, _K)
    valid_ref[...] = valid.astype(jnp.int8).reshape(1, 1, _K)
    s8_ref[...] = sup.reshape(1, _K, _K)


def _loop_kernel(s8_ref, valid_ref, msort_ref, keep_ref, scores_ref):
    s8 = s8_ref[...]
    valid = valid_ref[...].astype(jnp.int32) > 0
    ms = msort_ref[...]

    def cond(c):
        return jnp.logical_not(c[1])

    def body(c):
        k, _ = c
        s = lax.dot_general(k, s8, (((2,), (1,)), ((0,), (0,))),
                            preferred_element_type=jnp.int32)
        k_new = ((s == 0) & valid).astype(jnp.int8)
        done = jnp.sum(jnp.abs(k_new.astype(jnp.int32)
                               - k.astype(jnp.int32))) == 0
        return k_new, done

    k0 = valid.astype(jnp.int8)
    k_fix, _ = lax.while_loop(cond, body, (k0, jnp.bool_(False)))
    keep = k_fix.astype(jnp.int32) > 0

    keep_ref[...] = keep[..., :_TOPK].astype(jnp.int32)
    scores_ref[...] = jnp.where(keep, ms, 0.0)[..., :_TOPK]


@jax.jit
def kernel(raw_cls_bbox, raw_prob):
    scores_t = jnp.pad(raw_prob[:, 1:].T, ((0, 0), (0, _NPAD - _N)))
    scores3 = scores_t.reshape(_C, _ROWS, _LANES)
    boxes = raw_cls_bbox.reshape(_N, _C + 1, 4)[:, 1:, :]
    boxes_t = jnp.pad(jnp.transpose(boxes, (1, 2, 0)),
                      ((0, 0), (0, 0), (0, _NPAD - _N)))

    pos = pl.pallas_call(
        _select_kernel,
        out_shape=jax.ShapeDtypeStruct((_C, _ROWS, _LANES), jnp.int32),
    )(scores3)

    a6 = _sc_compact(pos.reshape(_C, _NPAD), scores_t, boxes_t)
    a6_t = jnp.transpose(a6, (0, 2, 1))

    top_boxes, labels, msort, valid8, s8 = pl.pallas_call(
        _nms_kernel,
        grid=(_C,),
        in_specs=[
            pl.BlockSpec((1, 6, _K), lambda c: (c, 0, 0)),
            pl.BlockSpec((1, _K, 6), lambda c: (c, 0, 0)),
        ],
        out_specs=[
            pl.BlockSpec((1, _TOPK, 4), lambda c: (c, 0, 0)),
            pl.BlockSpec((1, 1, _TOPK), lambda c: (c, 0, 0)),
            pl.BlockSpec((1, 1, _K), lambda c: (c, 0, 0)),
            pl.BlockSpec((1, 1, _K), lambda c: (c, 0, 0)),
            pl.BlockSpec((1, _K, _K), lambda c: (c, 0, 0)),
        ],
        out_shape=[
            jax.ShapeDtypeStruct((_C, _TOPK, 4), jnp.float32),
            jax.ShapeDtypeStruct((_C, 1, _TOPK), jnp.int32),
            jax.ShapeDtypeStruct((_C, 1, _K), jnp.float32),
            jax.ShapeDtypeStruct((_C, 1, _K), jnp.int8),
            jax.ShapeDtypeStruct((_C, _K, _K), jnp.int8),
        ],
    )(a6, a6_t)

    keep_i32, out_scores = pl.pallas_call(
        _loop_kernel,
        out_shape=[
            jax.ShapeDtypeStruct((_C, 1, _TOPK), jnp.int32),
            jax.ShapeDtypeStruct((_C, 1, _TOPK), jnp.float32),
        ],
    )(s8, valid8, msort)

    return (top_boxes, out_scores.reshape(_C, _TOPK),
            labels.reshape(_C, _TOPK),
            keep_i32.reshape(_C, _TOPK).astype(bool))


# speedup vs baseline: 6.1154x; 1.0271x over previous
import functools

import jax
import jax.numpy as jnp
from jax import lax
from jax.experimental import pallas as pl
from jax.experimental.pallas import tpu as pltpu
from jax.experimental.pallas import tpu_sc as plsc

_C = 20
_N = 20000
_NPAD = 20480
_ROWS = 160
_LANES = 128
_K = 1024
_TOPK = 1000
_NMS_T = 0.3
_SCORE_T = 0.05
_NCHUNK = 10
_CHUNK = 2048

_HI = jax.lax.Precision.HIGHEST


def _select_kernel(s_ref, pos_ref):
    s = s_ref[...]
    m = jnp.where(s > _SCORE_T, s, 0.0)
    key = lax.bitcast_convert_type(m, jnp.int32)
    ii = (lax.broadcasted_iota(jnp.int32, (_C, _ROWS, _LANES), 1) * _LANES
          + lax.broadcasted_iota(jnp.int32, (_C, _ROWS, _LANES), 2))

    def bs_body(_, c):
        lo, hi = c
        mid = lo + (hi - lo) // 2
        cnt = jnp.sum((key >= mid).astype(jnp.int32), axis=(1, 2),
                      keepdims=True)
        pred = cnt >= _K
        return jnp.where(pred, mid, lo), jnp.where(pred, hi, mid)

    lo0 = jnp.zeros((_C, 1, 1), jnp.int32)
    hi0 = jnp.full((_C, 1, 1), 0x7FFFFFFF, jnp.int32)
    vstar, _ = lax.fori_loop(0, 31, bs_body, (lo0, hi0))

    c_gt = jnp.sum((key > vstar).astype(jnp.int32), axis=(1, 2),
                   keepdims=True)
    need_eq = _K - c_gt
    eq = key == vstar

    def ts_body(_, c):
        lo, hi = c
        mid = lo + (hi - lo) // 2
        cnt = jnp.sum((eq & (ii < mid)).astype(jnp.int32), axis=(1, 2),
                      keepdims=True)
        pred = cnt >= need_eq
        return jnp.where(pred, lo, mid), jnp.where(pred, mid, hi)

    lo0 = jnp.zeros((_C, 1, 1), jnp.int32)
    hi0 = jnp.full((_C, 1, 1), _NPAD, jnp.int32)
    _, tstar = lax.fori_loop(0, 15, ts_body, (lo0, hi0))

    mask = (key > vstar) | (eq & (ii < tstar))
    mf = mask.astype(jnp.float32)

    u128 = (lax.broadcasted_iota(jnp.int32, (_LANES, _LANES), 0)
            < lax.broadcasted_iota(jnp.int32, (_LANES, _LANES), 1))
    lane_ex = lax.dot_general(mf.reshape(_C * _ROWS, _LANES),
                              u128.astype(jnp.float32),
                              (((1,), (0,)), ((), ())), precision=_HI,
                              preferred_element_type=jnp.float32)
    lane_ex = lane_ex.reshape(_C, _ROWS, _LANES)
    rowsum = jnp.sum(mf, axis=2, keepdims=True)
    rs = jnp.concatenate(
        [jnp.zeros((_C, 1, 1), jnp.float32), rowsum[:, :-1, :]], axis=1)
    d = 1
    while d < _ROWS:
        shifted = jnp.concatenate(
            [jnp.zeros((_C, d, 1), jnp.float32), rs[:, :-d, :]], axis=1)
        rs = rs + shifted
        d *= 2
    pos = lane_ex + rs
    pos_ref[...] = jnp.where(mask, pos.astype(jnp.int32), -1)


_NSLICE = _NPAD // 16
_SC_MESH = plsc.VectorSubcoreMesh(core_axis_name="c", subcore_axis_name="s")


@functools.partial(
    pl.kernel,
    mesh=_SC_MESH,
    out_type=jax.ShapeDtypeStruct((_C, 6, _K), jnp.float32),
    compiler_params=pltpu.CompilerParams(needs_layout_passes=False),
    scratch_types=[
        pltpu.VMEM((_NPAD,), jnp.int32),
        pltpu.VMEM((_NPAD,), jnp.float32),
        pltpu.VMEM((_K,), jnp.float32),
        pltpu.VMEM((_K,), jnp.float32),
        pltpu.VMEM((_K,), jnp.float32),
        pltpu.VMEM((_K,), jnp.float32),
        pltpu.VMEM((_K,), jnp.float32),
    ],
)
def _sc_compact(pos_hbm, s_hbm, b_hbm, out_hbm, pos_v, val_v,
                a0, a1, a2, a3, a4):
    wid = lax.axis_index("s") * 2 + lax.axis_index("c")
    outs = (a0, a1, a2, a3, a4)

    @pl.when(wid < _C)
    def _():
        pltpu.sync_copy(pos_hbm.at[wid], pos_v)
        for k in range(5):
            if k == 0:
                pltpu.sync_copy(s_hbm.at[wid], val_v)
            else:
                pltpu.sync_copy(b_hbm.at[wid, k - 1], val_v)

            def body(i, carry, k=k):
                idx = pos_v[pl.ds(i * 16, 16)]
                v16 = val_v[pl.ds(i * 16, 16)]
                if k == 0:
                    v16 = jnp.where(v16 > _SCORE_T, v16, 0.0)
                plsc.store_scatter(outs[k], [idx], v16, mask=idx >= 0)
                return carry

            lax.fori_loop(0, _NSLICE, body, 0)
        for k in range(5):
            pltpu.sync_copy(outs[k], out_hbm.at[wid, k])


def _nms_kernel(a_ref, at_ref, boxes_ref, labels_ref, keep_ref, scores_ref,
                s8_scr, valid_scr, ms_scr):
    a = a_ref[...].reshape(6, _K)
    at = at_ref[...].reshape(_K, 6)
    m_row = a[0:1, :]
    m_col = at[:, 0:1]
    q_row = lax.broadcasted_iota(jnp.int32, (1, _K), 1)
    p_col = lax.broadcasted_iota(jnp.int32, (_K, 1), 0)

    cmp = (m_row > m_col) | ((m_row == m_col) & (q_row < p_col))
    rank = jnp.sum(cmp.astype(jnp.int32), axis=1, keepdims=True)
    r_row = lax.broadcasted_iota(jnp.int32, (1, _K), 1)
    onehot2 = (rank == r_row).astype(jnp.float32)

    sorted_row = lax.dot_general(a[0:5, :], onehot2, (((1,), (0,)), ((), ())),
                                 precision=_HI,
                                 preferred_element_type=jnp.float32)
    sorted_col = lax.dot_general(onehot2, at[:, 0:5],
                                 (((0,), (0,)), ((), ())), precision=_HI,
                                 preferred_element_type=jnp.float32)

    ms_row = sorted_row[0:1, :]
    y1r, x1r = sorted_row[1:2, :], sorted_row[2:3, :]
    y2r, x2r = sorted_row[3:4, :], sorted_row[4:5, :]
    y1c, x1c = sorted_col[:, 1:2], sorted_col[:, 2:3]
    y2c, x2c = sorted_col[:, 3:4], sorted_col[:, 4:5]

    r_i32 = lax.broadcasted_iota(jnp.int32, (1, _K), 1)
    valid = (ms_row > _SCORE_T) & (r_i32 < _TOPK)

    yy1 = jnp.maximum(y1c, y1r)
    xx1 = jnp.maximum(x1c, x1r)
    yy2 = jnp.minimum(y2c, y2r)
    xx2 = jnp.minimum(x2c, x2r)
    inter = (jnp.clip(yy2 - yy1, 0.0) * jnp.clip(xx2 - xx1, 0.0))
    area_c = (y2c - y1c) * (x2c - x1c)
    area_r = (y2r - y1r) * (x2r - x1r)
    union = area_c + area_r - inter
    iou = inter / jnp.maximum(union, 1e-9)

    i_col = lax.broadcasted_iota(jnp.int32, (_K, _K), 0)
    j_row = lax.broadcasted_iota(jnp.int32, (_K, _K), 1)
    sup = ((iou > _NMS_T) & (j_row > i_col)).astype(jnp.int8)

    cls = pl.program_id(0)
    boxes_ref[...] = sorted_col[:_TOPK, 1:5].reshape(1, _TOPK, 4)
    labels_ref[...] = jnp.full((1, 1, _TOPK), cls, jnp.int32)
    s8_scr[cls] = sup
    valid_scr[cls] = valid.astype(jnp.int8)
    ms_scr[cls] = ms_row

    @pl.when(cls == _C - 1)
    def _():
        s8 = s8_scr[...]
        valid_all = valid_scr[...].astype(jnp.int32) > 0
        ms_all = ms_scr[...]

        def cond(c):
            return jnp.logical_not(c[1])

        def body(c):
            k, _ = c
            s = lax.dot_general(k, s8, (((2,), (1,)), ((0,), (0,))),
                                preferred_element_type=jnp.int32)
            k_new = ((s == 0) & valid_all).astype(jnp.int8)
            done = jnp.sum(jnp.abs(k_new.astype(jnp.int32)
                                   - k.astype(jnp.int32))) == 0
            return k_new, done

        k0 = valid_all.astype(jnp.int8)
        k_fix, _ = lax.while_loop(cond, body, (k0, jnp.bool_(False)))
        keep = k_fix.astype(jnp.int32) > 0

        keep_ref[...] = keep[..., :_TOPK].astype(jnp.int32)
        scores_ref[...] = jnp.where(keep, ms_all, 0.0)[..., :_TOPK]


@jax.jit
def kernel(raw_cls_bbox, raw_prob):
    scores_t = jnp.pad(raw_prob[:, 1:].T, ((0, 0), (0, _NPAD - _N)))
    scores3 = scores_t.reshape(_C, _ROWS, _LANES)
    boxes = raw_cls_bbox.reshape(_N, _C + 1, 4)[:, 1:, :]
    boxes_t = jnp.pad(jnp.transpose(boxes, (1, 2, 0)),
                      ((0, 0), (0, 0), (0, _NPAD - _N)))

    pos = pl.pallas_call(
        _select_kernel,
        out_shape=jax.ShapeDtypeStruct((_C, _ROWS, _LANES), jnp.int32),
    )(scores3)

    a6 = _sc_compact(pos.reshape(_C, _NPAD), scores_t, boxes_t)
    a6_t = jnp.transpose(a6, (0, 2, 1))

    top_boxes, labels, keep_i32, out_scores = pl.pallas_call(
        _nms_kernel,
        grid=(_C,),
        in_specs=[
            pl.BlockSpec((1, 6, _K), lambda c: (c, 0, 0)),
            pl.BlockSpec((1, _K, 6), lambda c: (c, 0, 0)),
        ],
        out_specs=[
            pl.BlockSpec((1, _TOPK, 4), lambda c: (c, 0, 0)),
            pl.BlockSpec((1, 1, _TOPK), lambda c: (c, 0, 0)),
            pl.BlockSpec((_C, 1, _TOPK), lambda c: (0, 0, 0)),
            pl.BlockSpec((_C, 1, _TOPK), lambda c: (0, 0, 0)),
        ],
        out_shape=[
            jax.ShapeDtypeStruct((_C, _TOPK, 4), jnp.float32),
            jax.ShapeDtypeStruct((_C, 1, _TOPK), jnp.int32),
            jax.ShapeDtypeStruct((_C, 1, _TOPK), jnp.int32),
            jax.ShapeDtypeStruct((_C, 1, _TOPK), jnp.float32),
        ],
        scratch_shapes=[
            pltpu.VMEM((_C, _K, _K), jnp.int8),
            pltpu.VMEM((_C, 1, _K), jnp.int8),
            pltpu.VMEM((_C, 1, _K), jnp.float32),
        ],
        compiler_params=pltpu.CompilerParams(
            vmem_limit_bytes=100 * 1024 * 1024),
    )(a6, a6_t)

    return (top_boxes, out_scores.reshape(_C, _TOPK),
            labels.reshape(_C, _TOPK),
            keep_i32.reshape(_C, _TOPK).astype(bool))


# speedup vs baseline: 6.1933x; 1.0127x over previous
import functools

import jax
import jax.numpy as jnp
from jax import lax
from jax.experimental import pallas as pl
from jax.experimental.pallas import tpu as pltpu
from jax.experimental.pallas import tpu_sc as plsc

_C = 20
_N = 20000
_NPAD = 20480
_ROWS = 160
_LANES = 128
_K = 1024
_TOPK = 1000
_NMS_T = 0.3
_SCORE_T = 0.05
_NCHUNK = 10
_CHUNK = 2048

_HI = jax.lax.Precision.HIGHEST


def _select_kernel(s_ref, pos_ref, m_ref):
    s = s_ref[...]
    m = jnp.where(s > _SCORE_T, s, 0.0)
    key = lax.bitcast_convert_type(m, jnp.int32)
    ii = (lax.broadcasted_iota(jnp.int32, (_C, _ROWS, _LANES), 1) * _LANES
          + lax.broadcasted_iota(jnp.int32, (_C, _ROWS, _LANES), 2))

    def bs_body(_, c):
        lo, hi = c
        mid = lo + (hi - lo) // 2
        cnt = jnp.sum((key >= mid).astype(jnp.int32), axis=(1, 2),
                      keepdims=True)
        pred = cnt >= _K
        return jnp.where(pred, mid, lo), jnp.where(pred, hi, mid)

    lo0 = jnp.zeros((_C, 1, 1), jnp.int32)
    hi0 = jnp.full((_C, 1, 1), 0x7FFFFFFF, jnp.int32)
    vstar, _ = lax.fori_loop(0, 31, bs_body, (lo0, hi0))

    c_gt = jnp.sum((key > vstar).astype(jnp.int32), axis=(1, 2),
                   keepdims=True)
    need_eq = _K - c_gt
    eq = key == vstar

    def ts_body(_, c):
        lo, hi = c
        mid = lo + (hi - lo) // 2
        cnt = jnp.sum((eq & (ii < mid)).astype(jnp.int32), axis=(1, 2),
                      keepdims=True)
        pred = cnt >= need_eq
        return jnp.where(pred, lo, mid), jnp.where(pred, mid, hi)

    lo0 = jnp.zeros((_C, 1, 1), jnp.int32)
    hi0 = jnp.full((_C, 1, 1), _NPAD, jnp.int32)
    _, tstar = lax.fori_loop(0, 15, ts_body, (lo0, hi0))

    mask = (key > vstar) | (eq & (ii < tstar))
    mf = mask.astype(jnp.float32)

    u128 = (lax.broadcasted_iota(jnp.int32, (_LANES, _LANES), 0)
            < lax.broadcasted_iota(jnp.int32, (_LANES, _LANES), 1))
    lane_ex = lax.dot_general(mf.reshape(_C * _ROWS, _LANES),
                              u128.astype(jnp.float32),
                              (((1,), (0,)), ((), ())), precision=_HI,
                              preferred_element_type=jnp.float32)
    lane_ex = lane_ex.reshape(_C, _ROWS, _LANES)
    rowsum = jnp.sum(mf, axis=2, keepdims=True)
    rs = jnp.concatenate(
        [jnp.zeros((_C, 1, 1), jnp.float32), rowsum[:, :-1, :]], axis=1)
    d = 1
    while d < _ROWS:
        shifted = jnp.concatenate(
            [jnp.zeros((_C, d, 1), jnp.float32), rs[:, :-d, :]], axis=1)
        rs = rs + shifted
        d *= 2
    pos = lane_ex + rs
    pos_ref[...] = jnp.where(mask, pos.astype(jnp.int32), -1)
    m_ref[...] = m


_NSLICE = _NPAD // 16
_SC_MESH = plsc.VectorSubcoreMesh(core_axis_name="c", subcore_axis_name="s")


_NTASK = _C * 5
_NWORKER = 32


@functools.partial(
    pl.kernel,
    mesh=_SC_MESH,
    out_type=jax.ShapeDtypeStruct((_C, 6, _K), jnp.float32),
    compiler_params=pltpu.CompilerParams(needs_layout_passes=False),
    scratch_types=[
        pltpu.VMEM((_NPAD,), jnp.int32),
        pltpu.VMEM((_NPAD,), jnp.float32),
        pltpu.VMEM((_K,), jnp.float32),
    ],
)
def _sc_compact(pos_hbm, vals_hbm, out_hbm, pos_v, val_v, a_v):
    wid = lax.axis_index("s") * 2 + lax.axis_index("c")

    for j in range((_NTASK + _NWORKER - 1) // _NWORKER):
        t = wid + _NWORKER * j

        @pl.when(t < _NTASK)
        def _(t=t):
            c = t // 5
            k = t - 5 * c
            pltpu.sync_copy(pos_hbm.at[c], pos_v)
            pltpu.sync_copy(vals_hbm.at[c, k], val_v)

            def body(i, carry):
                idx = pos_v[pl.ds(i * 16, 16)]
                v16 = val_v[pl.ds(i * 16, 16)]
                plsc.store_scatter(a_v, [idx], v16, mask=idx >= 0)
                return carry

            lax.fori_loop(0, _NSLICE, body, 0)
            pltpu.sync_copy(a_v, out_hbm.at[c, k])


def _nms_kernel(a_ref, at_ref, boxes_ref, labels_ref, keep_ref, scores_ref,
                s8_scr, valid_scr, ms_scr):
    a = a_ref[...].reshape(6, _K)
    at = at_ref[...].reshape(_K, 6)
    m_row = a[0:1, :]
    m_col = at[:, 0:1]
    q_row = lax.broadcasted_iota(jnp.int32, (1, _K), 1)
    p_col = lax.broadcasted_iota(jnp.int32, (_K, 1), 0)

    cmp = (m_row > m_col) | ((m_row == m_col) & (q_row < p_col))
    rank = jnp.sum(cmp.astype(jnp.int32), axis=1, keepdims=True)
    r_row = lax.broadcasted_iota(jnp.int32, (1, _K), 1)
    onehot2 = (rank == r_row).astype(jnp.float32)

    sorted_row = lax.dot_general(a[0:5, :], onehot2, (((1,), (0,)), ((), ())),
                                 precision=_HI,
                                 preferred_element_type=jnp.float32)
    sorted_col = lax.dot_general(onehot2, at[:, 0:5],
                                 (((0,), (0,)), ((), ())), precision=_HI,
                                 preferred_element_type=jnp.float32)

    ms_row = sorted_row[0:1, :]
    y1r, x1r = sorted_row[1:2, :], sorted_row[2:3, :]
    y2r, x2r = sorted_row[3:4, :], sorted_row[4:5, :]
    y1c, x1c = sorted_col[:, 1:2], sorted_col[:, 2:3]
    y2c, x2c = sorted_col[:, 3:4], sorted_col[:, 4:5]

    r_i32 = lax.broadcasted_iota(jnp.int32, (1, _K), 1)
    valid = (ms_row > _SCORE_T) & (r_i32 < _TOPK)

    yy1 = jnp.maximum(y1c, y1r)
    xx1 = jnp.maximum(x1c, x1r)
    yy2 = jnp.minimum(y2c, y2r)
    xx2 = jnp.minimum(x2c, x2r)
    inter = (jnp.clip(yy2 - yy1, 0.0) * jnp.clip(xx2 - xx1, 0.0))
    area_c = (y2c - y1c) * (x2c - x1c)
    area_r = (y2r - y1r) * (x2r - x1r)
    union = area_c + area_r - inter
    iou = inter / jnp.maximum(union, 1e-9)

    i_col = lax.broadcasted_iota(jnp.int32, (_K, _K), 0)
    j_row = lax.broadcasted_iota(jnp.int32, (_K, _K), 1)
    sup = ((iou > _NMS_T) & (j_row > i_col)).astype(jnp.int8)

    cls = pl.program_id(0)
    boxes_ref[...] = sorted_col[:_TOPK, 1:5].reshape(1, _TOPK, 4)
    labels_ref[...] = jnp.full((1, 1, _TOPK), cls, jnp.int32)
    s8_scr[cls] = sup
    valid_scr[cls] = valid.astype(jnp.int8)
    ms_scr[cls] = ms_row

    @pl.when(cls == _C - 1)
    def _():
        s8 = s8_scr[...]
        valid_all = valid_scr[...].astype(jnp.int32) > 0
        ms_all = ms_scr[...]

        def cond(c):
            return jnp.logical_not(c[1])

        def body(c):
            k, _ = c
            s = lax.dot_general(k, s8, (((2,), (1,)), ((0,), (0,))),
                                preferred_element_type=jnp.int32)
            k_new = ((s == 0) & valid_all).astype(jnp.int8)
            done = jnp.sum(jnp.abs(k_new.astype(jnp.int32)
                                   - k.astype(jnp.int32))) == 0
            return k_new, done

        k0 = valid_all.astype(jnp.int8)
        k_fix, _ = lax.while_loop(cond, body, (k0, jnp.bool_(False)))
        keep = k_fix.astype(jnp.int32) > 0

        keep_ref[...] = keep[..., :_TOPK].astype(jnp.int32)
        scores_ref[...] = jnp.where(keep, ms_all, 0.0)[..., :_TOPK]


@jax.jit
def kernel(raw_cls_bbox, raw_prob):
    scores_t = jnp.pad(raw_prob[:, 1:].T, ((0, 0), (0, _NPAD - _N)))
    scores3 = scores_t.reshape(_C, _ROWS, _LANES)
    boxes = raw_cls_bbox.reshape(_N, _C + 1, 4)[:, 1:, :]
    boxes_t = jnp.pad(jnp.transpose(boxes, (1, 2, 0)),
                      ((0, 0), (0, 0), (0, _NPAD - _N)))

    pos, m3 = pl.pallas_call(
        _select_kernel,
        out_shape=[
            jax.ShapeDtypeStruct((_C, _ROWS, _LANES), jnp.int32),
            jax.ShapeDtypeStruct((_C, _ROWS, _LANES), jnp.float32),
        ],
    )(scores3)

    vals = jnp.concatenate(
        [m3.reshape(_C, 1, _NPAD), boxes_t], axis=1)
    a6 = _sc_compact(pos.reshape(_C, _NPAD), vals)
    a6_t = jnp.transpose(a6, (0, 2, 1))

    top_boxes, labels, keep_i32, out_scores = pl.pallas_call(
        _nms_kernel,
        grid=(_C,),
        in_specs=[
            pl.BlockSpec((1, 6, _K), lambda c: (c, 0, 0)),
            pl.BlockSpec((1, _K, 6), lambda c: (c, 0, 0)),
        ],
        out_specs=[
            pl.BlockSpec((1, _TOPK, 4), lambda c: (c, 0, 0)),
            pl.BlockSpec((1, 1, _TOPK), lambda c: (c, 0, 0)),
            pl.BlockSpec((_C, 1, _TOPK), lambda c: (0, 0, 0)),
            pl.BlockSpec((_C, 1, _TOPK), lambda c: (0, 0, 0)),
        ],
        out_shape=[
            jax.ShapeDtypeStruct((_C, _TOPK, 4), jnp.float32),
            jax.ShapeDtypeStruct((_C, 1, _TOPK), jnp.int32),
            jax.ShapeDtypeStruct((_C, 1, _TOPK), jnp.int32),
            jax.ShapeDtypeStruct((_C, 1, _TOPK), jnp.float32),
        ],
        scratch_shapes=[
            pltpu.VMEM((_C, _K, _K), jnp.int8),
            pltpu.VMEM((_C, 1, _K), jnp.int8),
            pltpu.VMEM((_C, 1, _K), jnp.float32),
        ],
        compiler_params=pltpu.CompilerParams(
            vmem_limit_bytes=100 * 1024 * 1024),
    )(a6, a6_t)

    return (top_boxes, out_scores.reshape(_C, _TOPK),
            labels.reshape(_C, _TOPK),
            keep_i32.reshape(_C, _TOPK).astype(bool))


# speedup vs baseline: 6.8901x; 1.1125x over previous
import functools

import jax
import jax.numpy as jnp
from jax import lax
from jax.experimental import pallas as pl
from jax.experimental.pallas import tpu as pltpu
from jax.experimental.pallas import tpu_sc as plsc

_C = 20
_N = 20000
_NPAD = 20480
_ROWS = 160
_LANES = 128
_K = 1024
_TOPK = 1000
_NMS_T = 0.3
_SCORE_T = 0.05
_NCHUNK = 10
_CHUNK = 2048
_B = 128
_NB = _K // _B

_HI = jax.lax.Precision.HIGHEST


def _select_kernel(s_ref, pos_ref, m_ref):
    s = s_ref[...]
    m = jnp.where(s > _SCORE_T, s, 0.0)
    key = lax.bitcast_convert_type(m, jnp.int32)
    ii = (lax.broadcasted_iota(jnp.int32, (_C, _ROWS, _LANES), 1) * _LANES
          + lax.broadcasted_iota(jnp.int32, (_C, _ROWS, _LANES), 2))

    def bs_body(_, c):
        lo, hi = c
        mid = lo + (hi - lo) // 2
        cnt = jnp.sum((key >= mid).astype(jnp.int32), axis=(1, 2),
                      keepdims=True)
        pred = cnt >= _K
        return jnp.where(pred, mid, lo), jnp.where(pred, hi, mid)

    lo0 = jnp.zeros((_C, 1, 1), jnp.int32)
    hi0 = jnp.full((_C, 1, 1), 0x7FFFFFFF, jnp.int32)
    vstar, _ = lax.fori_loop(0, 31, bs_body, (lo0, hi0))

    c_gt = jnp.sum((key > vstar).astype(jnp.int32), axis=(1, 2),
                   keepdims=True)
    need_eq = _K - c_gt
    eq = key == vstar

    def ts_body(_, c):
        lo, hi = c
        mid = lo + (hi - lo) // 2
        cnt = jnp.sum((eq & (ii < mid)).astype(jnp.int32), axis=(1, 2),
                      keepdims=True)
        pred = cnt >= need_eq
        return jnp.where(pred, lo, mid), jnp.where(pred, mid, hi)

    lo0 = jnp.zeros((_C, 1, 1), jnp.int32)
    hi0 = jnp.full((_C, 1, 1), _NPAD, jnp.int32)
    _, tstar = lax.fori_loop(0, 15, ts_body, (lo0, hi0))

    mask = (key > vstar) | (eq & (ii < tstar))
    mf = mask.astype(jnp.float32)

    u128 = (lax.broadcasted_iota(jnp.int32, (_LANES, _LANES), 0)
            < lax.broadcasted_iota(jnp.int32, (_LANES, _LANES), 1))
    lane_ex = lax.dot_general(mf.reshape(_C * _ROWS, _LANES),
                              u128.astype(jnp.float32),
                              (((1,), (0,)), ((), ())), precision=_HI,
                              preferred_element_type=jnp.float32)
    lane_ex = lane_ex.reshape(_C, _ROWS, _LANES)
    rowsum = jnp.sum(mf, axis=2, keepdims=True)
    rs = jnp.concatenate(
        [jnp.zeros((_C, 1, 1), jnp.float32), rowsum[:, :-1, :]], axis=1)
    d = 1
    while d < _ROWS:
        shifted = jnp.concatenate(
            [jnp.zeros((_C, d, 1), jnp.float32), rs[:, :-d, :]], axis=1)
        rs = rs + shifted
        d *= 2
    pos = lane_ex + rs
    pos_ref[...] = jnp.where(mask, pos.astype(jnp.int32), -1)
    m_ref[...] = m


_NSLICE = _NPAD // 16
_SC_MESH = plsc.VectorSubcoreMesh(core_axis_name="c", subcore_axis_name="s")


_NTASK = _C * 5
_NWORKER = 32


@functools.partial(
    pl.kernel,
    mesh=_SC_MESH,
    out_type=jax.ShapeDtypeStruct((_C, 6, _K), jnp.float32),
    compiler_params=pltpu.CompilerParams(needs_layout_passes=False),
    scratch_types=[
        pltpu.VMEM((_NPAD,), jnp.int32),
        pltpu.VMEM((_NPAD,), jnp.float32),
        pltpu.VMEM((_K,), jnp.float32),
    ],
)
def _sc_compact(pos_hbm, vals_hbm, out_hbm, pos_v, val_v, a_v):
    wid = lax.axis_index("s") * 2 + lax.axis_index("c")

    for j in range((_NTASK + _NWORKER - 1) // _NWORKER):
        t = wid + _NWORKER * j

        @pl.when(t < _NTASK)
        def _(t=t):
            c = t // 5
            k = t - 5 * c
            pltpu.sync_copy(pos_hbm.at[c], pos_v)
            pltpu.sync_copy(vals_hbm.at[c, k], val_v)

            @plsc.parallel_loop(0, _NSLICE, 1, unroll=8)
            def body(i):
                idx = pos_v[pl.ds(i * 16, 16)]
                v16 = val_v[pl.ds(i * 16, 16)]
                plsc.store_scatter(a_v, [idx], v16, mask=idx >= 0)

            pltpu.sync_copy(a_v, out_hbm.at[c, k])


def _nms_kernel(a_ref, at_ref, boxes_ref, labels_ref, keep_ref, scores_ref,
                s8_scr, valid_scr, ms_scr):
    a = a_ref[...].reshape(6, _K)
    at = at_ref[...].reshape(_K, 6)
    m_row = a[0:1, :]
    m_col = at[:, 0:1]
    q_row = lax.broadcasted_iota(jnp.int32, (1, _K), 1)
    p_col = lax.broadcasted_iota(jnp.int32, (_K, 1), 0)

    cmp = (m_row > m_col) | ((m_row == m_col) & (q_row < p_col))
    rank = jnp.sum(cmp.astype(jnp.int32), axis=1, keepdims=True)
    r_row = lax.broadcasted_iota(jnp.int32, (1, _K), 1)
    onehot2 = (rank == r_row).astype(jnp.float32)

    sorted_row = lax.dot_general(a[0:5, :], onehot2, (((1,), (0,)), ((), ())),
                                 precision=_HI,
                                 preferred_element_type=jnp.float32)
    sorted_col = lax.dot_general(onehot2, at[:, 0:5],
                                 (((0,), (0,)), ((), ())), precision=_HI,
                                 preferred_element_type=jnp.float32)

    ms_row = sorted_row[0:1, :]
    y1r, x1r = sorted_row[1:2, :], sorted_row[2:3, :]
    y2r, x2r = sorted_row[3:4, :], sorted_row[4:5, :]
    y1c, x1c = sorted_col[:, 1:2], sorted_col[:, 2:3]
    y2c, x2c = sorted_col[:, 3:4], sorted_col[:, 4:5]

    r_i32 = lax.broadcasted_iota(jnp.int32, (1, _K), 1)
    valid = (ms_row > _SCORE_T) & (r_i32 < _TOPK)

    area_c = (y2c - y1c) * (x2c - x1c)
    area_r = (y2r - y1r) * (x2r - x1r)

    cls = pl.program_id(0)
    s8_scr[cls] = jnp.zeros((_K, _K), jnp.int8)
    ib = lax.broadcasted_iota(jnp.int32, (_B, _B), 0)
    jb = lax.broadcasted_iota(jnp.int32, (_B, _B), 1)
    for bi in range(_NB):
        r0 = bi * _B
        y1cb, x1cb = y1c[r0:r0 + _B, :], x1c[r0:r0 + _B, :]
        y2cb, x2cb = y2c[r0:r0 + _B, :], x2c[r0:r0 + _B, :]
        acb = area_c[r0:r0 + _B, :]
        for bj in range(bi, _NB):
            c0 = bj * _B
            yy1 = jnp.maximum(y1cb, y1r[:, c0:c0 + _B])
            xx1 = jnp.maximum(x1cb, x1r[:, c0:c0 + _B])
            yy2 = jnp.minimum(y2cb, y2r[:, c0:c0 + _B])
            xx2 = jnp.minimum(x2cb, x2r[:, c0:c0 + _B])
            inter = (jnp.clip(yy2 - yy1, 0.0) * jnp.clip(xx2 - xx1, 0.0))
            union = acb + area_r[:, c0:c0 + _B] - inter
            iou = inter / jnp.maximum(union, 1e-9)
            supb = iou > _NMS_T
            if bi == bj:
                supb = supb & (jb > ib)
            s8_scr[cls, r0:r0 + _B, c0:c0 + _B] = supb.astype(jnp.int8)

    boxes_ref[...] = sorted_col[:_TOPK, 1:5].reshape(1, _TOPK, 4)
    labels_ref[...] = jnp.full((1, 1, _TOPK), cls, jnp.int32)
    valid_scr[cls] = valid.astype(jnp.int8)
    ms_scr[cls] = ms_row

    @pl.when(cls == _C - 1)
    def _():
        s8 = s8_scr[...]
        valid_all = valid_scr[...].astype(jnp.int32) > 0
        ms_all = ms_scr[...]

        def cond(c):
            return jnp.logical_not(c[1])

        def body(c):
            k, _ = c
            s = lax.dot_general(k, s8, (((2,), (1,)), ((0,), (0,))),
                                preferred_element_type=jnp.int32)
            k_new = ((s == 0) & valid_all).astype(jnp.int8)
            done = jnp.sum(jnp.abs(k_new.astype(jnp.int32)
                                   - k.astype(jnp.int32))) == 0
            return k_new, done

        k0 = valid_all.astype(jnp.int8)
        k_fix, _ = lax.while_loop(cond, body, (k0, jnp.bool_(False)))
        keep = k_fix.astype(jnp.int32) > 0

        keep_ref[...] = keep[..., :_TOPK].astype(jnp.int32)
        scores_ref[...] = jnp.where(keep, ms_all, 0.0)[..., :_TOPK]


@jax.jit
def kernel(raw_cls_bbox, raw_prob):
    scores_t = jnp.pad(raw_prob[:, 1:].T, ((0, 0), (0, _NPAD - _N)))
    scores3 = scores_t.reshape(_C, _ROWS, _LANES)
    boxes = raw_cls_bbox.reshape(_N, _C + 1, 4)[:, 1:, :]
    boxes_t = jnp.pad(jnp.transpose(boxes, (1, 2, 0)),
                      ((0, 0), (0, 0), (0, _NPAD - _N)))

    pos, m3 = pl.pallas_call(
        _select_kernel,
        out_shape=[
            jax.ShapeDtypeStruct((_C, _ROWS, _LANES), jnp.int32),
            jax.ShapeDtypeStruct((_C, _ROWS, _LANES), jnp.float32),
        ],
    )(scores3)

    vals = jnp.concatenate(
        [m3.reshape(_C, 1, _NPAD), boxes_t], axis=1)
    a6 = _sc_compact(pos.reshape(_C, _NPAD), vals)
    a6_t = jnp.transpose(a6, (0, 2, 1))

    top_boxes, labels, keep_i32, out_scores = pl.pallas_call(
        _nms_kernel,
        grid=(_C,),
        in_specs=[
            pl.BlockSpec((1, 6, _K), lambda c: (c, 0, 0)),
            pl.BlockSpec((1, _K, 6), lambda c: (c, 0, 0)),
        ],
        out_specs=[
            pl.BlockSpec((1, _TOPK, 4), lambda c: (c, 0, 0)),
            pl.BlockSpec((1, 1, _TOPK), lambda c: (c, 0, 0)),
            pl.BlockSpec((_C, 1, _TOPK), lambda c: (0, 0, 0)),
            pl.BlockSpec((_C, 1, _TOPK), lambda c: (0, 0, 0)),
        ],
        out_shape=[
            jax.ShapeDtypeStruct((_C, _TOPK, 4), jnp.float32),
            jax.ShapeDtypeStruct((_C, 1, _TOPK), jnp.int32),
            jax.ShapeDtypeStruct((_C, 1, _TOPK), jnp.int32),
            jax.ShapeDtypeStruct((_C, 1, _TOPK), jnp.float32),
        ],
        scratch_shapes=[
            pltpu.VMEM((_C, _K, _K), jnp.int8),
            pltpu.VMEM((_C, 1, _K), jnp.int8),
            pltpu.VMEM((_C, 1, _K), jnp.float32),
        ],
        compiler_params=pltpu.CompilerParams(
            vmem_limit_bytes=100 * 1024 * 1024),
    )(a6, a6_t)

    return (top_boxes, out_scores.reshape(_C, _TOPK),
            labels.reshape(_C, _TOPK),
            keep_i32.reshape(_C, _TOPK).astype(bool))
